# Initial kernel scaffold; baseline (speedup 1.0000x reference)
import jax
import jax.numpy as jnp
from jax.experimental import pallas as pl


def kernel(x, edge_index, batch, Wl1, bl1, Wr1, Wl2, bl2, Wr2, Wl3, bl3, Wr3, Wl4, bl4, Wr4):
    raise NotImplementedError("write your pallas kernel here")


# speedup vs baseline: 5.8681x; 5.8681x over previous
import functools

import jax
import jax.numpy as jnp
from jax import lax
from jax.experimental import pallas as pl
from jax.experimental.pallas import tpu as pltpu
from jax.experimental.pallas import tpu_sc as plsc

N = 10000
E = 320000
D = 128
G = 64

NPAD = 10240
NC = 2
NS = 16
NW = NC * NS
CH = 128
NCHUNK = E // CH
CHUNK_ITERS = -(-NCHUNK // NW)
ROWS_PER_TILE = NPAD // NS
DEGW = 128

RB = 1024
GRID = NPAD // RB

_sc_mesh = plsc.VectorSubcoreMesh(core_axis_name="c", subcore_axis_name="s")


def _sc_agg_body(h_hbm, src_hbm, dst_hbm, out_hbm, acc, zbuf, sidx, didx, rows, sem):
    c = lax.axis_index("c")
    s = lax.axis_index("s")
    wid = s * NC + c

    @pl.loop(0, 16)
    def _(i):
        for j in range(8):
            zbuf[i, pl.ds(j * 16, 16)] = jnp.zeros((16,), jnp.float32)

    base = s * ROWS_PER_TILE

    @pl.loop(0, ROWS_PER_TILE // 16)
    def _(k):
        pltpu.sync_copy(zbuf, acc.at[pl.ds(base + k * 16, 16)])

    plsc.subcore_barrier()

    @pl.loop(0, CHUNK_ITERS)
    def _(i):
        cid = i * NW + wid

        @pl.when(cid < NCHUNK)
        def _():
            eb = cid * CH
            pltpu.sync_copy(src_hbm.at[pl.ds(eb, CH)], sidx)
            pltpu.sync_copy(dst_hbm.at[pl.ds(eb, CH)], didx)
            pltpu.async_copy(h_hbm.at[sidx], rows, sem).wait()
            pltpu.sync_copy(rows, acc.at[didx], add=True)

    plsc.subcore_barrier()
    pltpu.sync_copy(acc.at[pl.ds(base, ROWS_PER_TILE)],
                    out_hbm.at[c].at[pl.ds(base, ROWS_PER_TILE)])


def _sc_agg(h, src, dst):
    return pl.kernel(
        _sc_agg_body,
        out_type=jax.ShapeDtypeStruct((NC, NPAD, D), jnp.float32),
        mesh=_sc_mesh,
        scratch_types=[
            pltpu.VMEM_SHARED((NPAD, D), jnp.float32),
            pltpu.VMEM((16, D), jnp.float32),
            pltpu.VMEM((CH,), jnp.int32),
            pltpu.VMEM((CH,), jnp.int32),
            pltpu.VMEM((CH, D), jnp.float32),
            pltpu.SemaphoreType.DMA,
        ],
    )(h, src, dst)


def _sc_deg_body(dst_hbm, out_hbm, dacc, zbuf, ones_v, didx):
    c = lax.axis_index("c")
    s = lax.axis_index("s")
    wid = s * NC + c

    @pl.loop(0, 16)
    def _(i):
        for j in range(DEGW // 16):
            zbuf[i, pl.ds(j * 16, 16)] = jnp.zeros((16,), jnp.float32)

    @pl.loop(0, CH)
    def _(i):
        for j in range(DEGW // 16):
            ones_v[i, pl.ds(j * 16, 16)] = jnp.ones((16,), jnp.float32)

    base = s * ROWS_PER_TILE

    @pl.loop(0, ROWS_PER_TILE // 16)
    def _(k):
        pltpu.sync_copy(zbuf, dacc.at[pl.ds(base + k * 16, 16)])

    plsc.subcore_barrier()

    @pl.loop(0, CHUNK_ITERS)
    def _(i):
        cid = i * NW + wid

        @pl.when(cid < NCHUNK)
        def _():
            pltpu.sync_copy(dst_hbm.at[pl.ds(cid * CH, CH)], didx)
            pltpu.sync_copy(ones_v, dacc.at[didx], add=True)

    plsc.subcore_barrier()
    pltpu.sync_copy(dacc.at[pl.ds(base, ROWS_PER_TILE)],
                    out_hbm.at[c].at[pl.ds(base, ROWS_PER_TILE)])


def _sc_deg(dst):
    return pl.kernel(
        _sc_deg_body,
        out_type=jax.ShapeDtypeStruct((NC, NPAD, DEGW), jnp.float32),
        mesh=_sc_mesh,
        scratch_types=[
            pltpu.VMEM_SHARED((NPAD, DEGW), jnp.float32),
            pltpu.VMEM((16, DEGW), jnp.float32),
            pltpu.VMEM((CH, DEGW), jnp.float32),
            pltpu.VMEM((CH,), jnp.int32),
        ],
    )(dst)


def _tc_layer_body(ap_ref, dp_ref, h_ref, wl_ref, bl_ref, wr_ref, o_ref, *, relu):
    a = ap_ref[0] + ap_ref[1]
    deg = dp_ref[0, :, 0:1] + dp_ref[1, :, 0:1]
    inv = 1.0 / jnp.maximum(deg, 1.0)
    agg = a * inv
    out = lax.dot_general(agg, wl_ref[...], (((1,), (0,)), ((), ())),
                          precision=lax.Precision.HIGHEST,
                          preferred_element_type=jnp.float32)
    out += lax.dot_general(h_ref[...], wr_ref[...], (((1,), (0,)), ((), ())),
                           precision=lax.Precision.HIGHEST,
                           preferred_element_type=jnp.float32)
    out += bl_ref[...]
    if relu:
        out = jnp.maximum(out, 0.0)
    o_ref[...] = out


def _tc_layer(parts, degp, h, Wl, bl, Wr, relu):
    return pl.pallas_call(
        functools.partial(_tc_layer_body, relu=relu),
        grid=(GRID,),
        in_specs=[
            pl.BlockSpec((NC, RB, D), lambda i: (0, i, 0)),
            pl.BlockSpec((NC, RB, DEGW), lambda i: (0, i, 0)),
            pl.BlockSpec((RB, D), lambda i: (i, 0)),
            pl.BlockSpec((D, D), lambda i: (0, 0)),
            pl.BlockSpec((1, D), lambda i: (0, 0)),
            pl.BlockSpec((D, D), lambda i: (0, 0)),
        ],
        out_specs=pl.BlockSpec((RB, D), lambda i: (i, 0)),
        out_shape=jax.ShapeDtypeStruct((NPAD, D), jnp.float32),
    )(parts, degp, h, Wl, bl.reshape(1, D), Wr)


def _tc_pool_body(h_ref, b_ref, o_ref, s_acc, c_acc):
    i = pl.program_id(0)

    @pl.when(i == 0)
    def _():
        s_acc[...] = jnp.zeros_like(s_acc)
        c_acc[...] = jnp.zeros_like(c_acc)

    b = b_ref[0, 0, :]
    onehot = (lax.broadcasted_iota(jnp.int32, (G, RB), 0) == b[None, :]).astype(jnp.float32)
    s_acc[...] += lax.dot_general(onehot, h_ref[...], (((1,), (0,)), ((), ())),
                                  precision=lax.Precision.HIGHEST,
                                  preferred_element_type=jnp.float32)
    cnt = jnp.sum(onehot, axis=1, keepdims=True)
    c_acc[...] += jnp.broadcast_to(cnt, (G, D))

    @pl.when(i == GRID - 1)
    def _():
        o_ref[...] = s_acc[...] / jnp.maximum(c_acc[...], 1.0)


def _tc_pool(h, batch_r):
    return pl.pallas_call(
        _tc_pool_body,
        grid=(GRID,),
        in_specs=[
            pl.BlockSpec((RB, D), lambda i: (i, 0)),
            pl.BlockSpec((1, 1, RB), lambda i: (i, 0, 0)),
        ],
        out_specs=pl.BlockSpec((G, D), lambda i: (0, 0)),
        out_shape=jax.ShapeDtypeStruct((G, D), jnp.float32),
        scratch_shapes=[
            pltpu.VMEM((G, D), jnp.float32),
            pltpu.VMEM((G, D), jnp.float32),
        ],
    )(h, batch_r)


def kernel(x, edge_index, batch, Wl1, bl1, Wr1, Wl2, bl2, Wr2, Wl3, bl3, Wr3,
           Wl4, bl4, Wr4):
    src = edge_index[0]
    dst = edge_index[1]
    h = jnp.pad(x, ((0, NPAD - N), (0, 0)))
    batch_r = jnp.pad(batch, (0, NPAD - N), constant_values=G).reshape(GRID, 1, RB)

    degp = _sc_deg(dst)

    for Wl, bl, Wr, relu in ((Wl1, bl1, Wr1, True), (Wl2, bl2, Wr2, True),
                             (Wl3, bl3, Wr3, True), (Wl4, bl4, Wr4, False)):
        parts = _sc_agg(h, src, dst)
        h = _tc_layer(parts, degp, h, Wl, bl, Wr, relu)

    return _tc_pool(h, batch_r)


# speedup vs baseline: 10.1199x; 1.7246x over previous
import functools

import jax
import jax.numpy as jnp
from jax import lax
from jax.experimental import pallas as pl
from jax.experimental.pallas import tpu as pltpu
from jax.experimental.pallas import tpu_sc as plsc

N = 10000
E = 320000
D = 128
G = 64

NPAD = 10240
NC = 2
NS = 16
NW = NC * NS
CH = 125
NCHUNK = E // CH
CPT = NCHUNK // NW
ROWS_PER_TILE = NPAD // NS
DEGW = 128

RB = 1024
GRID = NPAD // RB

_sc_mesh = plsc.VectorSubcoreMesh(core_axis_name="c", subcore_axis_name="s")


def _zero_tile_slice(acc, zbuf, base, nrows):
    @pl.loop(0, 16)
    def _(i):
        for j in range(zbuf.shape[1] // 16):
            zbuf[i, pl.ds(j * 16, 16)] = jnp.zeros((16,), jnp.float32)

    @pl.loop(0, nrows // 16)
    def _(k):
        pltpu.sync_copy(zbuf, acc.at[pl.ds(base + k * 16, 16)])


def _sc_agg_body(h_hbm, src_hbm, dst_hbm, out_hbm, acc, zbuf, sidx, dring, rows,
                 sg0, sg1, ss0, ss1, si0, si1):
    c = lax.axis_index("c")
    s = lax.axis_index("s")
    wid = s * NC + c
    cbase = wid * CPT

    pltpu.sync_copy(src_hbm.at[pl.ds(cbase, CPT)], sidx)
    pltpu.async_copy(dst_hbm.at[cbase], dring.at[0], si0)
    pltpu.async_copy(dst_hbm.at[cbase + 1], dring.at[1], si1)
    pltpu.async_copy(h_hbm.at[sidx.at[0]], rows.at[0], sg0)

    _zero_tile_slice(acc, zbuf, s * ROWS_PER_TILE, ROWS_PER_TILE)
    plsc.subcore_barrier()

    sg = (sg0, sg1)
    ss = (ss0, ss1)
    si = (si0, si1)

    @pl.loop(0, CPT // 2)
    def _(i):
        for b in range(2):
            k = i * 2 + b
            r = lax.rem(k, 4)
            pltpu.make_async_copy(dst_hbm.at[cbase], dring.at[r], si[b]).wait()
            pltpu.make_async_copy(h_hbm.at[sidx.at[k]], rows.at[b], sg[b]).wait()
            pltpu.async_copy(rows.at[b], acc.at[dring.at[r]], ss[b], add=True)
            o = 1 - b
            if b == 0:
                @pl.when(i > 0)
                def _():
                    pltpu.make_async_copy(rows.at[o], acc.at[dring.at[r]], ss[o]).wait()

                pltpu.async_copy(h_hbm.at[sidx.at[k + 1]], rows.at[o], sg[o])
            else:
                @pl.when(i < CPT // 2 - 1)
                def _():
                    pltpu.make_async_copy(rows.at[o], acc.at[dring.at[r]], ss[o]).wait()
                    pltpu.async_copy(h_hbm.at[sidx.at[k + 1]], rows.at[o], sg[o])

            @pl.when(k < CPT - 2)
            def _():
                pltpu.async_copy(dst_hbm.at[cbase + k + 2],
                                 dring.at[lax.rem(k + 2, 4)], si[b])

    pltpu.make_async_copy(rows.at[0], acc.at[dring.at[0]], ss0).wait()
    pltpu.make_async_copy(rows.at[1], acc.at[dring.at[0]], ss1).wait()

    plsc.subcore_barrier()
    base = s * ROWS_PER_TILE
    pltpu.sync_copy(acc.at[pl.ds(base, ROWS_PER_TILE)],
                    out_hbm.at[c].at[pl.ds(base, ROWS_PER_TILE)])


def _sc_agg(h, src_r, dst_r):
    return pl.kernel(
        _sc_agg_body,
        out_type=jax.ShapeDtypeStruct((NC, NPAD, D), jnp.float32),
        mesh=_sc_mesh,
        scratch_types=[
            pltpu.VMEM_SHARED((NPAD, D), jnp.float32),
            pltpu.VMEM((16, D), jnp.float32),
            pltpu.VMEM((CPT, CH), jnp.int32),
            pltpu.VMEM((4, CH), jnp.int32),
            pltpu.VMEM((2, CH, D), jnp.float32),
            pltpu.SemaphoreType.DMA,
            pltpu.SemaphoreType.DMA,
            pltpu.SemaphoreType.DMA,
            pltpu.SemaphoreType.DMA,
            pltpu.SemaphoreType.DMA,
            pltpu.SemaphoreType.DMA,
        ],
    )(h, src_r, dst_r)


def _sc_deg_body(dst_hbm, out_hbm, dacc, zbuf, ones_v, didx, ss0, ss1):
    c = lax.axis_index("c")
    s = lax.axis_index("s")
    wid = s * NC + c

    pltpu.sync_copy(dst_hbm.at[pl.ds(wid * CPT, CPT)], didx)

    @pl.loop(0, CH)
    def _(i):
        for j in range(DEGW // 16):
            ones_v[i, pl.ds(j * 16, 16)] = jnp.ones((16,), jnp.float32)

    _zero_tile_slice(dacc, zbuf, s * ROWS_PER_TILE, ROWS_PER_TILE)
    plsc.subcore_barrier()

    ss = (ss0, ss1)

    @pl.loop(0, CPT // 2)
    def _(i):
        for b in range(2):
            k = i * 2 + b

            @pl.when(i > 0)
            def _():
                pltpu.make_async_copy(ones_v, dacc.at[didx.at[k]], ss[b]).wait()

            pltpu.async_copy(ones_v, dacc.at[didx.at[k]], ss[b], add=True)

    pltpu.make_async_copy(ones_v, dacc.at[didx.at[0]], ss0).wait()
    pltpu.make_async_copy(ones_v, dacc.at[didx.at[0]], ss1).wait()

    plsc.subcore_barrier()
    base = s * ROWS_PER_TILE
    pltpu.sync_copy(dacc.at[pl.ds(base, ROWS_PER_TILE)],
                    out_hbm.at[c].at[pl.ds(base, ROWS_PER_TILE)])


def _sc_deg(dst_r):
    return pl.kernel(
        _sc_deg_body,
        out_type=jax.ShapeDtypeStruct((NC, NPAD, DEGW), jnp.float32),
        mesh=_sc_mesh,
        scratch_types=[
            pltpu.VMEM_SHARED((NPAD, DEGW), jnp.float32),
            pltpu.VMEM((16, DEGW), jnp.float32),
            pltpu.VMEM((CH, DEGW), jnp.float32),
            pltpu.VMEM((CPT, CH), jnp.int32),
            pltpu.SemaphoreType.DMA,
            pltpu.SemaphoreType.DMA,
        ],
    )(dst_r)


def _tc_layer_body(ap_ref, dp_ref, h_ref, wl_ref, bl_ref, wr_ref, o_ref, *, relu):
    a = ap_ref[0] + ap_ref[1]
    deg = dp_ref[0, :, 0:1] + dp_ref[1, :, 0:1]
    inv = 1.0 / jnp.maximum(deg, 1.0)
    agg = a * inv
    out = lax.dot_general(agg, wl_ref[...], (((1,), (0,)), ((), ())),
                          precision=lax.Precision.HIGHEST,
                          preferred_element_type=jnp.float32)
    out += lax.dot_general(h_ref[...], wr_ref[...], (((1,), (0,)), ((), ())),
                           precision=lax.Precision.HIGHEST,
                           preferred_element_type=jnp.float32)
    out += bl_ref[...]
    if relu:
        out = jnp.maximum(out, 0.0)
    o_ref[...] = out


def _tc_layer(parts, degp, h, Wl, bl, Wr, relu):
    return pl.pallas_call(
        functools.partial(_tc_layer_body, relu=relu),
        grid=(GRID,),
        in_specs=[
            pl.BlockSpec((NC, RB, D), lambda i: (0, i, 0)),
            pl.BlockSpec((NC, RB, DEGW), lambda i: (0, i, 0)),
            pl.BlockSpec((RB, D), lambda i: (i, 0)),
            pl.BlockSpec((D, D), lambda i: (0, 0)),
            pl.BlockSpec((1, D), lambda i: (0, 0)),
            pl.BlockSpec((D, D), lambda i: (0, 0)),
        ],
        out_specs=pl.BlockSpec((RB, D), lambda i: (i, 0)),
        out_shape=jax.ShapeDtypeStruct((NPAD, D), jnp.float32),
    )(parts, degp, h, Wl, bl.reshape(1, D), Wr)


def _tc_pool_body(h_ref, b_ref, o_ref, s_acc, c_acc):
    i = pl.program_id(0)

    @pl.when(i == 0)
    def _():
        s_acc[...] = jnp.zeros_like(s_acc)
        c_acc[...] = jnp.zeros_like(c_acc)

    b = b_ref[0, 0, :]
    onehot = (lax.broadcasted_iota(jnp.int32, (G, RB), 0) == b[None, :]).astype(jnp.float32)
    s_acc[...] += lax.dot_general(onehot, h_ref[...], (((1,), (0,)), ((), ())),
                                  precision=lax.Precision.HIGHEST,
                                  preferred_element_type=jnp.float32)
    cnt = jnp.sum(onehot, axis=1, keepdims=True)
    c_acc[...] += jnp.broadcast_to(cnt, (G, D))

    @pl.when(i == GRID - 1)
    def _():
        o_ref[...] = s_acc[...] / jnp.maximum(c_acc[...], 1.0)


def _tc_pool(h, batch_r):
    return pl.pallas_call(
        _tc_pool_body,
        grid=(GRID,),
        in_specs=[
            pl.BlockSpec((RB, D), lambda i: (i, 0)),
            pl.BlockSpec((1, 1, RB), lambda i: (i, 0, 0)),
        ],
        out_specs=pl.BlockSpec((G, D), lambda i: (0, 0)),
        out_shape=jax.ShapeDtypeStruct((G, D), jnp.float32),
        scratch_shapes=[
            pltpu.VMEM((G, D), jnp.float32),
            pltpu.VMEM((G, D), jnp.float32),
        ],
    )(h, batch_r)


def kernel(x, edge_index, batch, Wl1, bl1, Wr1, Wl2, bl2, Wr2, Wl3, bl3, Wr3,
           Wl4, bl4, Wr4):
    src_r = edge_index[0].reshape(NCHUNK, CH)
    dst_r = edge_index[1].reshape(NCHUNK, CH)
    h = jnp.pad(x, ((0, NPAD - N), (0, 0)))
    batch_r = jnp.pad(batch, (0, NPAD - N), constant_values=G).reshape(GRID, 1, RB)

    degp = _sc_deg(dst_r)

    for Wl, bl, Wr, relu in ((Wl1, bl1, Wr1, True), (Wl2, bl2, Wr2, True),
                             (Wl3, bl3, Wr3, True), (Wl4, bl4, Wr4, False)):
        parts = _sc_agg(h, src_r, dst_r)
        h = _tc_layer(parts, degp, h, Wl, bl, Wr, relu)

    return _tc_pool(h, batch_r)


# speedup vs baseline: 10.2183x; 1.0097x over previous
import functools

import jax
import jax.numpy as jnp
from jax import lax
from jax.experimental import pallas as pl
from jax.experimental.pallas import tpu as pltpu
from jax.experimental.pallas import tpu_sc as plsc

N = 10000
E = 320000
D = 128
G = 64

NPAD = 10240
NC = 2
NS = 16
NW = NC * NS
CH = 125
NCHUNK = E // CH
CPT = NCHUNK // NW
ROWS_PER_TILE = NPAD // NS
DEGW = 128

RB = 1024
GRID = NPAD // RB

_sc_mesh = plsc.VectorSubcoreMesh(core_axis_name="c", subcore_axis_name="s")


def _zero_tile_slice(acc, zbuf, base, nrows, sem):
    zr = zbuf.shape[0]

    @pl.loop(0, zr)
    def _(i):
        for j in range(zbuf.shape[1] // 16):
            zbuf[i, pl.ds(j * 16, 16)] = jnp.zeros((16,), jnp.float32)

    pltpu.async_copy(zbuf, acc.at[pl.ds(base, zr)], sem)
    pltpu.async_copy(zbuf, acc.at[pl.ds(base + zr, zr)], sem)

    @pl.loop(2, nrows // zr)
    def _(k):
        pltpu.make_async_copy(zbuf, acc.at[pl.ds(base, zr)], sem).wait()
        pltpu.async_copy(zbuf, acc.at[pl.ds(base + k * zr, zr)], sem)

    pltpu.make_async_copy(zbuf, acc.at[pl.ds(base, zr)], sem).wait()
    pltpu.make_async_copy(zbuf, acc.at[pl.ds(base, zr)], sem).wait()


def _sc_agg_body(h_hbm, src_hbm, dst_hbm, out_hbm, acc, zbuf, sidx, dring, rows,
                 sg0, sg1, ss0, ss1, si0, si1, sz):
    c = lax.axis_index("c")
    s = lax.axis_index("s")
    wid = s * NC + c
    cbase = wid * CPT

    pltpu.sync_copy(src_hbm.at[pl.ds(cbase, CPT)], sidx)
    pltpu.async_copy(dst_hbm.at[cbase], dring.at[0], si0)
    pltpu.async_copy(dst_hbm.at[cbase + 1], dring.at[1], si1)
    pltpu.async_copy(h_hbm.at[sidx.at[0]], rows.at[0], sg0)

    _zero_tile_slice(acc, zbuf, s * ROWS_PER_TILE, ROWS_PER_TILE, sz)
    plsc.subcore_barrier()

    sg = (sg0, sg1)
    ss = (ss0, ss1)
    si = (si0, si1)

    @pl.loop(0, CPT // 2)
    def _(i):
        for b in range(2):
            k = i * 2 + b
            r = lax.rem(k, 4)
            pltpu.make_async_copy(dst_hbm.at[cbase], dring.at[r], si[b]).wait()
            pltpu.make_async_copy(h_hbm.at[sidx.at[k]], rows.at[b], sg[b]).wait()
            pltpu.async_copy(rows.at[b], acc.at[dring.at[r]], ss[b], add=True)
            o = 1 - b
            if b == 0:
                @pl.when(i > 0)
                def _():
                    pltpu.make_async_copy(rows.at[o], acc.at[dring.at[r]], ss[o]).wait()

                pltpu.async_copy(h_hbm.at[sidx.at[k + 1]], rows.at[o], sg[o])
            else:
                @pl.when(i < CPT // 2 - 1)
                def _():
                    pltpu.make_async_copy(rows.at[o], acc.at[dring.at[r]], ss[o]).wait()
                    pltpu.async_copy(h_hbm.at[sidx.at[k + 1]], rows.at[o], sg[o])

            @pl.when(k < CPT - 2)
            def _():
                pltpu.async_copy(dst_hbm.at[cbase + k + 2],
                                 dring.at[lax.rem(k + 2, 4)], si[b])

    pltpu.make_async_copy(rows.at[0], acc.at[dring.at[0]], ss0).wait()
    pltpu.make_async_copy(rows.at[1], acc.at[dring.at[0]], ss1).wait()

    plsc.subcore_barrier()
    base = s * ROWS_PER_TILE
    pltpu.sync_copy(acc.at[pl.ds(base, ROWS_PER_TILE)],
                    out_hbm.at[c].at[pl.ds(base, ROWS_PER_TILE)])


def _sc_agg(h, src_r, dst_r):
    return pl.kernel(
        _sc_agg_body,
        out_type=jax.ShapeDtypeStruct((NC, NPAD, D), jnp.float32),
        mesh=_sc_mesh,
        scratch_types=[
            pltpu.VMEM_SHARED((NPAD, D), jnp.float32),
            pltpu.VMEM((32, D), jnp.float32),
            pltpu.VMEM((CPT, CH), jnp.int32),
            pltpu.VMEM((4, CH), jnp.int32),
            pltpu.VMEM((2, CH, D), jnp.float32),
            pltpu.SemaphoreType.DMA,
            pltpu.SemaphoreType.DMA,
            pltpu.SemaphoreType.DMA,
            pltpu.SemaphoreType.DMA,
            pltpu.SemaphoreType.DMA,
            pltpu.SemaphoreType.DMA,
            pltpu.SemaphoreType.DMA,
        ],
    )(h, src_r, dst_r)


def _sc_deg_body(dst_hbm, out_hbm, dacc, zbuf, ones_v, didx, ss0, ss1, sz):
    c = lax.axis_index("c")
    s = lax.axis_index("s")
    wid = s * NC + c

    pltpu.sync_copy(dst_hbm.at[pl.ds(wid * CPT, CPT)], didx)

    @pl.loop(0, CH)
    def _(i):
        for j in range(DEGW // 16):
            ones_v[i, pl.ds(j * 16, 16)] = jnp.ones((16,), jnp.float32)

    _zero_tile_slice(dacc, zbuf, s * ROWS_PER_TILE, ROWS_PER_TILE, sz)
    plsc.subcore_barrier()

    ss = (ss0, ss1)

    @pl.loop(0, CPT // 2)
    def _(i):
        for b in range(2):
            k = i * 2 + b

            @pl.when(i > 0)
            def _():
                pltpu.make_async_copy(ones_v, dacc.at[didx.at[k]], ss[b]).wait()

            pltpu.async_copy(ones_v, dacc.at[didx.at[k]], ss[b], add=True)

    pltpu.make_async_copy(ones_v, dacc.at[didx.at[0]], ss0).wait()
    pltpu.make_async_copy(ones_v, dacc.at[didx.at[0]], ss1).wait()

    plsc.subcore_barrier()
    base = s * ROWS_PER_TILE
    pltpu.sync_copy(dacc.at[pl.ds(base, ROWS_PER_TILE)],
                    out_hbm.at[c].at[pl.ds(base, ROWS_PER_TILE)])


def _sc_deg(dst_r):
    return pl.kernel(
        _sc_deg_body,
        out_type=jax.ShapeDtypeStruct((NC, NPAD, DEGW), jnp.float32),
        mesh=_sc_mesh,
        scratch_types=[
            pltpu.VMEM_SHARED((NPAD, DEGW), jnp.float32),
            pltpu.VMEM((32, DEGW), jnp.float32),
            pltpu.VMEM((CH, DEGW), jnp.float32),
            pltpu.VMEM((CPT, CH), jnp.int32),
            pltpu.SemaphoreType.DMA,
            pltpu.SemaphoreType.DMA,
            pltpu.SemaphoreType.DMA,
        ],
    )(dst_r)


def _tc_layer_body(ap_ref, dp_ref, h_ref, wl_ref, bl_ref, wr_ref, o_ref, *, relu):
    a = ap_ref[0] + ap_ref[1]
    deg = dp_ref[0, :, 0:1] + dp_ref[1, :, 0:1]
    inv = 1.0 / jnp.maximum(deg, 1.0)
    agg = a * inv
    out = lax.dot_general(agg, wl_ref[...], (((1,), (0,)), ((), ())),
                          precision=lax.Precision.HIGHEST,
                          preferred_element_type=jnp.float32)
    out += lax.dot_general(h_ref[...], wr_ref[...], (((1,), (0,)), ((), ())),
                           precision=lax.Precision.HIGHEST,
                           preferred_element_type=jnp.float32)
    out += bl_ref[...]
    if relu:
        out = jnp.maximum(out, 0.0)
    o_ref[...] = out


def _tc_layer(parts, degp, h, Wl, bl, Wr, relu):
    return pl.pallas_call(
        functools.partial(_tc_layer_body, relu=relu),
        grid=(GRID,),
        in_specs=[
            pl.BlockSpec((NC, RB, D), lambda i: (0, i, 0)),
            pl.BlockSpec((NC, RB, DEGW), lambda i: (0, i, 0)),
            pl.BlockSpec((RB, D), lambda i: (i, 0)),
            pl.BlockSpec((D, D), lambda i: (0, 0)),
            pl.BlockSpec((1, D), lambda i: (0, 0)),
            pl.BlockSpec((D, D), lambda i: (0, 0)),
        ],
        out_specs=pl.BlockSpec((RB, D), lambda i: (i, 0)),
        out_shape=jax.ShapeDtypeStruct((NPAD, D), jnp.float32),
    )(parts, degp, h, Wl, bl.reshape(1, D), Wr)


def _tc_pool_body(h_ref, b_ref, o_ref, s_acc, c_acc):
    i = pl.program_id(0)

    @pl.when(i == 0)
    def _():
        s_acc[...] = jnp.zeros_like(s_acc)
        c_acc[...] = jnp.zeros_like(c_acc)

    b = b_ref[0, 0, :]
    onehot = (lax.broadcasted_iota(jnp.int32, (G, RB), 0) == b[None, :]).astype(jnp.float32)
    s_acc[...] += lax.dot_general(onehot, h_ref[...], (((1,), (0,)), ((), ())),
                                  precision=lax.Precision.HIGHEST,
                                  preferred_element_type=jnp.float32)
    cnt = jnp.sum(onehot, axis=1, keepdims=True)
    c_acc[...] += jnp.broadcast_to(cnt, (G, D))

    @pl.when(i == GRID - 1)
    def _():
        o_ref[...] = s_acc[...] / jnp.maximum(c_acc[...], 1.0)


def _tc_pool(h, batch_r):
    return pl.pallas_call(
        _tc_pool_body,
        grid=(GRID,),
        in_specs=[
            pl.BlockSpec((RB, D), lambda i: (i, 0)),
            pl.BlockSpec((1, 1, RB), lambda i: (i, 0, 0)),
        ],
        out_specs=pl.BlockSpec((G, D), lambda i: (0, 0)),
        out_shape=jax.ShapeDtypeStruct((G, D), jnp.float32),
        scratch_shapes=[
            pltpu.VMEM((G, D), jnp.float32),
            pltpu.VMEM((G, D), jnp.float32),
        ],
    )(h, batch_r)


def kernel(x, edge_index, batch, Wl1, bl1, Wr1, Wl2, bl2, Wr2, Wl3, bl3, Wr3,
           Wl4, bl4, Wr4):
    src_r = edge_index[0].reshape(NCHUNK, CH)
    dst_r = edge_index[1].reshape(NCHUNK, CH)
    h = jnp.pad(x, ((0, NPAD - N), (0, 0)))
    batch_r = jnp.pad(batch, (0, NPAD - N), constant_values=G).reshape(GRID, 1, RB)

    degp = _sc_deg(dst_r)

    for Wl, bl, Wr, relu in ((Wl1, bl1, Wr1, True), (Wl2, bl2, Wr2, True),
                             (Wl3, bl3, Wr3, True), (Wl4, bl4, Wr4, False)):
        parts = _sc_agg(h, src_r, dst_r)
        h = _tc_layer(parts, degp, h, Wl, bl, Wr, relu)

    return _tc_pool(h, batch_r)


# speedup vs baseline: 10.4049x; 1.0183x over previous
import functools

import jax
import jax.numpy as jnp
from jax import lax
from jax.experimental import pallas as pl
from jax.experimental.pallas import tpu as pltpu
from jax.experimental.pallas import tpu_sc as plsc

N = 10000
E = 320000
D = 128
G = 64

NPAD = 10240
NC = 2
NS = 16
NW = NC * NS
CH = 125
NCHUNK = E // CH
CPT = NCHUNK // NW
ROWS_PER_TILE = NPAD // NS
DEGW = 128

RB = 1024
GRID = NPAD // RB

_sc_mesh = plsc.VectorSubcoreMesh(core_axis_name="c", subcore_axis_name="s")


def _zero_tile_slice(acc, zbuf, base, nrows, sem):
    zr = zbuf.shape[0]

    @pl.loop(0, zr)
    def _(i):
        for j in range(zbuf.shape[1] // 16):
            zbuf[i, pl.ds(j * 16, 16)] = jnp.zeros((16,), jnp.float32)

    pltpu.async_copy(zbuf, acc.at[pl.ds(base, zr)], sem)
    pltpu.async_copy(zbuf, acc.at[pl.ds(base + zr, zr)], sem)

    @pl.loop(2, nrows // zr)
    def _(k):
        pltpu.make_async_copy(zbuf, acc.at[pl.ds(base, zr)], sem).wait()
        pltpu.async_copy(zbuf, acc.at[pl.ds(base + k * zr, zr)], sem)

    pltpu.make_async_copy(zbuf, acc.at[pl.ds(base, zr)], sem).wait()
    pltpu.make_async_copy(zbuf, acc.at[pl.ds(base, zr)], sem).wait()


def _sc_agg_body(h_hbm, src_hbm, dst_hbm, out_hbm, acc, zbuf, sidx, dring, rows,
                 sg0, sg1, ss0, ss1, si0, si1, sz):
    c = lax.axis_index("c")
    s = lax.axis_index("s")
    wid = s * NC + c
    cbase = wid * CPT

    pltpu.sync_copy(src_hbm.at[pl.ds(cbase, CPT)], sidx)
    pltpu.async_copy(dst_hbm.at[cbase], dring.at[0], si0)
    pltpu.async_copy(dst_hbm.at[cbase + 1], dring.at[1], si1)
    pltpu.async_copy(h_hbm.at[sidx.at[0]], rows.at[0], sg0)

    _zero_tile_slice(acc, zbuf, s * ROWS_PER_TILE, ROWS_PER_TILE, sz)
    plsc.subcore_barrier()

    sg = (sg0, sg1)
    ss = (ss0, ss1)
    si = (si0, si1)

    @pl.loop(0, CPT // 2)
    def _(i):
        for b in range(2):
            k = i * 2 + b
            r = lax.rem(k, 4)
            pltpu.make_async_copy(dst_hbm.at[cbase], dring.at[r], si[b]).wait()
            pltpu.make_async_copy(h_hbm.at[sidx.at[k]], rows.at[b], sg[b]).wait()
            pltpu.async_copy(rows.at[b], acc.at[dring.at[r]], ss[b], add=True)
            o = 1 - b
            if b == 0:
                @pl.when(i > 0)
                def _():
                    pltpu.make_async_copy(rows.at[o], acc.at[dring.at[r]], ss[o]).wait()

                pltpu.async_copy(h_hbm.at[sidx.at[k + 1]], rows.at[o], sg[o])
            else:
                @pl.when(i < CPT // 2 - 1)
                def _():
                    pltpu.make_async_copy(rows.at[o], acc.at[dring.at[r]], ss[o]).wait()
                    pltpu.async_copy(h_hbm.at[sidx.at[k + 1]], rows.at[o], sg[o])

            @pl.when(k < CPT - 2)
            def _():
                pltpu.async_copy(dst_hbm.at[cbase + k + 2],
                                 dring.at[lax.rem(k + 2, 4)], si[b])

    pltpu.make_async_copy(rows.at[0], acc.at[dring.at[0]], ss0).wait()
    pltpu.make_async_copy(rows.at[1], acc.at[dring.at[0]], ss1).wait()

    plsc.subcore_barrier()
    base = s * ROWS_PER_TILE
    pltpu.sync_copy(acc.at[pl.ds(base, ROWS_PER_TILE)],
                    out_hbm.at[c].at[pl.ds(base, ROWS_PER_TILE)])


def _sc_agg(h, src_r, dst_r):
    return pl.kernel(
        _sc_agg_body,
        out_type=jax.ShapeDtypeStruct((NC, NPAD, D), jnp.float32),
        mesh=_sc_mesh,
        scratch_types=[
            pltpu.VMEM_SHARED((NPAD, D), jnp.float32),
            pltpu.VMEM((32, D), jnp.float32),
            pltpu.VMEM((CPT, CH), jnp.int32),
            pltpu.VMEM((4, CH), jnp.int32),
            pltpu.VMEM((2, CH, D), jnp.float32),
            pltpu.SemaphoreType.DMA,
            pltpu.SemaphoreType.DMA,
            pltpu.SemaphoreType.DMA,
            pltpu.SemaphoreType.DMA,
            pltpu.SemaphoreType.DMA,
            pltpu.SemaphoreType.DMA,
            pltpu.SemaphoreType.DMA,
        ],
    )(h, src_r, dst_r)


def _sc_deg_body(dst_hbm, out_hbm, dacc, zbuf, ones_v, didx, ss0, ss1, sz):
    c = lax.axis_index("c")
    s = lax.axis_index("s")
    wid = s * NC + c

    pltpu.sync_copy(dst_hbm.at[pl.ds(wid * CPT, CPT)], didx)

    @pl.loop(0, CH)
    def _(i):
        for j in range(DEGW // 16):
            ones_v[i, pl.ds(j * 16, 16)] = jnp.ones((16,), jnp.float32)

    _zero_tile_slice(dacc, zbuf, s * ROWS_PER_TILE, ROWS_PER_TILE, sz)
    plsc.subcore_barrier()

    ss = (ss0, ss1)

    @pl.loop(0, CPT // 2)
    def _(i):
        for b in range(2):
            k = i * 2 + b

            @pl.when(i > 0)
            def _():
                pltpu.make_async_copy(ones_v, dacc.at[didx.at[k]], ss[b]).wait()

            pltpu.async_copy(ones_v, dacc.at[didx.at[k]], ss[b], add=True)

    pltpu.make_async_copy(ones_v, dacc.at[didx.at[0]], ss0).wait()
    pltpu.make_async_copy(ones_v, dacc.at[didx.at[0]], ss1).wait()

    plsc.subcore_barrier()
    base = s * ROWS_PER_TILE
    pltpu.sync_copy(dacc.at[pl.ds(base, ROWS_PER_TILE)],
                    out_hbm.at[c].at[pl.ds(base, ROWS_PER_TILE)])


def _sc_deg(dst_r):
    return pl.kernel(
        _sc_deg_body,
        out_type=jax.ShapeDtypeStruct((NC, NPAD, DEGW), jnp.float32),
        mesh=_sc_mesh,
        scratch_types=[
            pltpu.VMEM_SHARED((NPAD, DEGW), jnp.float32),
            pltpu.VMEM((32, DEGW), jnp.float32),
            pltpu.VMEM((CH, DEGW), jnp.float32),
            pltpu.VMEM((CPT, CH), jnp.int32),
            pltpu.SemaphoreType.DMA,
            pltpu.SemaphoreType.DMA,
            pltpu.SemaphoreType.DMA,
        ],
    )(dst_r)


def _tc_r_body(h_ref, wr_ref, bl_ref, o_ref):
    o_ref[...] = lax.dot_general(h_ref[...], wr_ref[...], (((1,), (0,)), ((), ())),
                                 precision=lax.Precision.HIGHEST,
                                 preferred_element_type=jnp.float32) + bl_ref[...]


def _tc_r(h, Wr, bl):
    return pl.pallas_call(
        _tc_r_body,
        grid=(GRID,),
        in_specs=[
            pl.BlockSpec((RB, D), lambda i: (i, 0)),
            pl.BlockSpec((D, D), lambda i: (0, 0)),
            pl.BlockSpec((1, D), lambda i: (0, 0)),
        ],
        out_specs=pl.BlockSpec((RB, D), lambda i: (i, 0)),
        out_shape=jax.ShapeDtypeStruct((NPAD, D), jnp.float32),
    )(h, Wr, bl.reshape(1, D))


def _scaled_agg_matmul(ap_ref, dp_ref, wl_ref):
    a = ap_ref[0] + ap_ref[1]
    deg = dp_ref[0, :, 0:1] + dp_ref[1, :, 0:1]
    inv = 1.0 / jnp.maximum(deg, 1.0)
    return lax.dot_general(a * inv, wl_ref[...], (((1,), (0,)), ((), ())),
                           precision=lax.Precision.HIGHEST,
                           preferred_element_type=jnp.float32)


def _tc_combine_body(ap_ref, dp_ref, r_ref, wl_ref, o_ref):
    out = _scaled_agg_matmul(ap_ref, dp_ref, wl_ref) + r_ref[...]
    o_ref[...] = jnp.maximum(out, 0.0)


def _tc_combine(parts, degp, r, Wl):
    return pl.pallas_call(
        _tc_combine_body,
        grid=(GRID,),
        in_specs=[
            pl.BlockSpec((NC, RB, D), lambda i: (0, i, 0)),
            pl.BlockSpec((NC, RB, DEGW), lambda i: (0, i, 0)),
            pl.BlockSpec((RB, D), lambda i: (i, 0)),
            pl.BlockSpec((D, D), lambda i: (0, 0)),
        ],
        out_specs=pl.BlockSpec((RB, D), lambda i: (i, 0)),
        out_shape=jax.ShapeDtypeStruct((NPAD, D), jnp.float32),
    )(parts, degp, r, Wl)


def _tc_combine_pool_body(ap_ref, dp_ref, r_ref, wl_ref, b_ref, o_ref,
                          s_acc, c_acc):
    i = pl.program_id(0)

    @pl.when(i == 0)
    def _():
        s_acc[...] = jnp.zeros_like(s_acc)
        c_acc[...] = jnp.zeros_like(c_acc)

    h4 = _scaled_agg_matmul(ap_ref, dp_ref, wl_ref) + r_ref[...]
    b = b_ref[0, 0, :]
    onehot = (lax.broadcasted_iota(jnp.int32, (G, RB), 0) == b[None, :]).astype(jnp.float32)
    s_acc[...] += lax.dot_general(onehot, h4, (((1,), (0,)), ((), ())),
                                  precision=lax.Precision.HIGHEST,
                                  preferred_element_type=jnp.float32)
    cnt = jnp.sum(onehot, axis=1, keepdims=True)
    c_acc[...] += jnp.broadcast_to(cnt, (G, D))

    @pl.when(i == GRID - 1)
    def _():
        o_ref[...] = s_acc[...] / jnp.maximum(c_acc[...], 1.0)


def _tc_combine_pool(parts, degp, r, Wl, batch_r):
    return pl.pallas_call(
        _tc_combine_pool_body,
        grid=(GRID,),
        in_specs=[
            pl.BlockSpec((NC, RB, D), lambda i: (0, i, 0)),
            pl.BlockSpec((NC, RB, DEGW), lambda i: (0, i, 0)),
            pl.BlockSpec((RB, D), lambda i: (i, 0)),
            pl.BlockSpec((D, D), lambda i: (0, 0)),
            pl.BlockSpec((1, 1, RB), lambda i: (i, 0, 0)),
        ],
        out_specs=pl.BlockSpec((G, D), lambda i: (0, 0)),
        out_shape=jax.ShapeDtypeStruct((G, D), jnp.float32),
        scratch_shapes=[
            pltpu.VMEM((G, D), jnp.float32),
            pltpu.VMEM((G, D), jnp.float32),
        ],
    )(parts, degp, r, Wl, batch_r)


def kernel(x, edge_index, batch, Wl1, bl1, Wr1, Wl2, bl2, Wr2, Wl3, bl3, Wr3,
           Wl4, bl4, Wr4):
    src_r = edge_index[0].reshape(NCHUNK, CH)
    dst_r = edge_index[1].reshape(NCHUNK, CH)
    h = jnp.pad(x, ((0, NPAD - N), (0, 0)))
    batch_r = jnp.pad(batch, (0, NPAD - N), constant_values=G).reshape(GRID, 1, RB)

    degp = _sc_deg(dst_r)

    for Wl, bl, Wr in ((Wl1, bl1, Wr1), (Wl2, bl2, Wr2), (Wl3, bl3, Wr3)):
        parts = _sc_agg(h, src_r, dst_r)
        r = _tc_r(h, Wr, bl)
        h = _tc_combine(parts, degp, r, Wl)

    parts = _sc_agg(h, src_r, dst_r)
    r = _tc_r(h, Wr4, bl4)
    return _tc_combine_pool(parts, degp, r, Wl4, batch_r)


# speedup vs baseline: 10.9804x; 1.0553x over previous
import dataclasses
import functools

import jax
import jax.numpy as jnp
from jax import lax
from jax.experimental import pallas as pl
from jax.experimental.pallas import tpu as pltpu
from jax.experimental.pallas import tpu_sc as plsc

N = 10000
E = 320000
D = 128
G = 64

NPAD = 10240
NC = 2
NS = 16
NW = NC * NS
CH = 125
NCHUNK = E // CH
CPT = NCHUNK // NW
ROWS_PER_TILE = NPAD // NS
EPT = E // NW

RB = 1024
GRID = NPAD // RB

_sc_mesh = plsc.VectorSubcoreMesh(core_axis_name="c", subcore_axis_name="s")

_sc_params = pltpu.CompilerParams()
if "needs_layout_passes" in pltpu.CompilerParams.__dataclass_fields__:
    _sc_params = dataclasses.replace(_sc_params, needs_layout_passes=False)


def _zero_tile_slice(acc, zbuf, base, nrows, sem):
    zr = zbuf.shape[0]

    @pl.loop(0, zr)
    def _(i):
        for j in range(zbuf.shape[1] // 16):
            zbuf[i, pl.ds(j * 16, 16)] = jnp.zeros((16,), jnp.float32)

    pltpu.async_copy(zbuf, acc.at[pl.ds(base, zr)], sem)
    pltpu.async_copy(zbuf, acc.at[pl.ds(base + zr, zr)], sem)

    @pl.loop(2, nrows // zr)
    def _(k):
        pltpu.make_async_copy(zbuf, acc.at[pl.ds(base, zr)], sem).wait()
        pltpu.async_copy(zbuf, acc.at[pl.ds(base + k * zr, zr)], sem)

    pltpu.make_async_copy(zbuf, acc.at[pl.ds(base, zr)], sem).wait()
    pltpu.make_async_copy(zbuf, acc.at[pl.ds(base, zr)], sem).wait()


def _sc_agg_body(h_hbm, src_hbm, dst_hbm, out_hbm, acc, zbuf, sidx, dring, rows,
                 sg0, sg1, ss0, ss1, si0, si1, sz):
    c = lax.axis_index("c")
    s = lax.axis_index("s")
    wid = s * NC + c
    cbase = wid * CPT

    pltpu.sync_copy(src_hbm.at[pl.ds(cbase, CPT)], sidx)
    pltpu.async_copy(dst_hbm.at[cbase], dring.at[0], si0)
    pltpu.async_copy(dst_hbm.at[cbase + 1], dring.at[1], si1)
    pltpu.async_copy(h_hbm.at[sidx.at[0]], rows.at[0], sg0)

    _zero_tile_slice(acc, zbuf, s * ROWS_PER_TILE, ROWS_PER_TILE, sz)
    plsc.subcore_barrier()

    sg = (sg0, sg1)
    ss = (ss0, ss1)
    si = (si0, si1)

    @pl.loop(0, CPT // 2)
    def _(i):
        for b in range(2):
            k = i * 2 + b
            r = lax.rem(k, 4)
            pltpu.make_async_copy(dst_hbm.at[cbase], dring.at[r], si[b]).wait()
            pltpu.make_async_copy(h_hbm.at[sidx.at[k]], rows.at[b], sg[b]).wait()
            pltpu.async_copy(rows.at[b], acc.at[dring.at[r]], ss[b], add=True)
            o = 1 - b
            if b == 0:
                @pl.when(i > 0)
                def _():
                    pltpu.make_async_copy(rows.at[o], acc.at[dring.at[r]], ss[o]).wait()

                pltpu.async_copy(h_hbm.at[sidx.at[k + 1]], rows.at[o], sg[o])
            else:
                @pl.when(i < CPT // 2 - 1)
                def _():
                    pltpu.make_async_copy(rows.at[o], acc.at[dring.at[r]], ss[o]).wait()
                    pltpu.async_copy(h_hbm.at[sidx.at[k + 1]], rows.at[o], sg[o])

            @pl.when(k < CPT - 2)
            def _():
                pltpu.async_copy(dst_hbm.at[cbase + k + 2],
                                 dring.at[lax.rem(k + 2, 4)], si[b])

    pltpu.make_async_copy(rows.at[0], acc.at[dring.at[0]], ss0).wait()
    pltpu.make_async_copy(rows.at[1], acc.at[dring.at[0]], ss1).wait()

    plsc.subcore_barrier()
    base = s * ROWS_PER_TILE
    pltpu.sync_copy(acc.at[pl.ds(base, ROWS_PER_TILE)],
                    out_hbm.at[c].at[pl.ds(base, ROWS_PER_TILE)])


def _sc_agg(h, src_r, dst_r):
    return pl.kernel(
        _sc_agg_body,
        out_type=jax.ShapeDtypeStruct((NC, NPAD, D), jnp.float32),
        mesh=_sc_mesh,
        scratch_types=[
            pltpu.VMEM_SHARED((NPAD, D), jnp.float32),
            pltpu.VMEM((32, D), jnp.float32),
            pltpu.VMEM((CPT, CH), jnp.int32),
            pltpu.VMEM((4, CH), jnp.int32),
            pltpu.VMEM((2, CH, D), jnp.float32),
            pltpu.SemaphoreType.DMA,
            pltpu.SemaphoreType.DMA,
            pltpu.SemaphoreType.DMA,
            pltpu.SemaphoreType.DMA,
            pltpu.SemaphoreType.DMA,
            pltpu.SemaphoreType.DMA,
            pltpu.SemaphoreType.DMA,
        ],
    )(h, src_r, dst_r)


def _sc_deg_body(dst_hbm, out_hbm, degt, didx):
    c = lax.axis_index("c")
    s = lax.axis_index("s")
    wid = s * NC + c

    pltpu.sync_copy(dst_hbm.at[pl.ds(wid * EPT, EPT)], didx)

    @pl.loop(0, NPAD // 128)
    def _(k):
        for j in range(8):
            degt[pl.ds(k * 128 + j * 16, 16)] = jnp.zeros((16,), jnp.float32)

    ones16 = jnp.ones((16,), jnp.float32)

    @pl.loop(0, EPT // 80)
    def _(i):
        for j in range(5):
            idx = didx[pl.ds(i * 80 + j * 16, 16)]
            plsc.addupdate_scatter(degt, [idx], ones16)

    pltpu.sync_copy(degt, out_hbm.at[wid])


def _sc_deg(dst):
    return pl.kernel(
        _sc_deg_body,
        out_type=jax.ShapeDtypeStruct((NW, NPAD), jnp.float32),
        mesh=_sc_mesh,
        scratch_types=[
            pltpu.VMEM((NPAD,), jnp.float32),
            pltpu.VMEM((EPT,), jnp.int32),
        ],
        compiler_params=_sc_params,
    )(dst)


def _tc_r_body(h_ref, wr_ref, bl_ref, o_ref):
    o_ref[...] = lax.dot_general(h_ref[...], wr_ref[...], (((1,), (0,)), ((), ())),
                                 precision=lax.Precision.HIGHEST,
                                 preferred_element_type=jnp.float32) + bl_ref[...]


def _tc_r(h, Wr, bl):
    return pl.pallas_call(
        _tc_r_body,
        grid=(GRID,),
        in_specs=[
            pl.BlockSpec((RB, D), lambda i: (i, 0)),
            pl.BlockSpec((D, D), lambda i: (0, 0)),
            pl.BlockSpec((1, D), lambda i: (0, 0)),
        ],
        out_specs=pl.BlockSpec((RB, D), lambda i: (i, 0)),
        out_shape=jax.ShapeDtypeStruct((NPAD, D), jnp.float32),
    )(h, Wr, bl.reshape(1, D))


def _scaled_agg_matmul(ap_ref, dp_ref, wl_ref):
    a = ap_ref[0] + ap_ref[1]
    deg = lax.dot_general(dp_ref[...], jnp.ones((NW, 1), jnp.float32),
                          (((0,), (0,)), ((), ())),
                          precision=lax.Precision.HIGHEST,
                          preferred_element_type=jnp.float32)
    inv = 1.0 / jnp.maximum(deg, 1.0)
    return lax.dot_general(a * inv, wl_ref[...], (((1,), (0,)), ((), ())),
                           precision=lax.Precision.HIGHEST,
                           preferred_element_type=jnp.float32)


def _tc_combine_body(ap_ref, dp_ref, r_ref, wl_ref, o_ref):
    out = _scaled_agg_matmul(ap_ref, dp_ref, wl_ref) + r_ref[...]
    o_ref[...] = jnp.maximum(out, 0.0)


def _tc_combine(parts, degp, r, Wl):
    return pl.pallas_call(
        _tc_combine_body,
        grid=(GRID,),
        in_specs=[
            pl.BlockSpec((NC, RB, D), lambda i: (0, i, 0)),
            pl.BlockSpec((NW, RB), lambda i: (0, i)),
            pl.BlockSpec((RB, D), lambda i: (i, 0)),
            pl.BlockSpec((D, D), lambda i: (0, 0)),
        ],
        out_specs=pl.BlockSpec((RB, D), lambda i: (i, 0)),
        out_shape=jax.ShapeDtypeStruct((NPAD, D), jnp.float32),
    )(parts, degp, r, Wl)


def _tc_combine_pool_body(ap_ref, dp_ref, r_ref, wl_ref, b_ref, o_ref,
                          s_acc, c_acc):
    i = pl.program_id(0)

    @pl.when(i == 0)
    def _():
        s_acc[...] = jnp.zeros_like(s_acc)
        c_acc[...] = jnp.zeros_like(c_acc)

    h4 = _scaled_agg_matmul(ap_ref, dp_ref, wl_ref) + r_ref[...]
    b = b_ref[0, 0, :]
    onehot = (lax.broadcasted_iota(jnp.int32, (G, RB), 0) == b[None, :]).astype(jnp.float32)
    s_acc[...] += lax.dot_general(onehot, h4, (((1,), (0,)), ((), ())),
                                  precision=lax.Precision.HIGHEST,
                                  preferred_element_type=jnp.float32)
    cnt = jnp.sum(onehot, axis=1, keepdims=True)
    c_acc[...] += jnp.broadcast_to(cnt, (G, D))

    @pl.when(i == GRID - 1)
    def _():
        o_ref[...] = s_acc[...] / jnp.maximum(c_acc[...], 1.0)


def _tc_combine_pool(parts, degp, r, Wl, batch_r):
    return pl.pallas_call(
        _tc_combine_pool_body,
        grid=(GRID,),
        in_specs=[
            pl.BlockSpec((NC, RB, D), lambda i: (0, i, 0)),
            pl.BlockSpec((NW, RB), lambda i: (0, i)),
            pl.BlockSpec((RB, D), lambda i: (i, 0)),
            pl.BlockSpec((D, D), lambda i: (0, 0)),
            pl.BlockSpec((1, 1, RB), lambda i: (i, 0, 0)),
        ],
        out_specs=pl.BlockSpec((G, D), lambda i: (0, 0)),
        out_shape=jax.ShapeDtypeStruct((G, D), jnp.float32),
        scratch_shapes=[
            pltpu.VMEM((G, D), jnp.float32),
            pltpu.VMEM((G, D), jnp.float32),
        ],
    )(parts, degp, r, Wl, batch_r)


def kernel(x, edge_index, batch, Wl1, bl1, Wr1, Wl2, bl2, Wr2, Wl3, bl3, Wr3,
           Wl4, bl4, Wr4):
    src_r = edge_index[0].reshape(NCHUNK, CH)
    dst_r = edge_index[1].reshape(NCHUNK, CH)
    h = jnp.pad(x, ((0, NPAD - N), (0, 0)))
    batch_r = jnp.pad(batch, (0, NPAD - N), constant_values=G).reshape(GRID, 1, RB)

    degp = _sc_deg(edge_index[1])

    for Wl, bl, Wr in ((Wl1, bl1, Wr1), (Wl2, bl2, Wr2), (Wl3, bl3, Wr3)):
        parts = _sc_agg(h, src_r, dst_r)
        r = _tc_r(h, Wr, bl)
        h = _tc_combine(parts, degp, r, Wl)

    parts = _sc_agg(h, src_r, dst_r)
    r = _tc_r(h, Wr4, bl4)
    return _tc_combine_pool(parts, degp, r, Wl4, batch_r)


# speedup vs baseline: 12.8159x; 1.1672x over previous
import dataclasses
import functools

import jax
import jax.numpy as jnp
from jax import lax
from jax.experimental import pallas as pl
from jax.experimental.pallas import tpu as pltpu
from jax.experimental.pallas import tpu_sc as plsc

N = 10000
E = 320000
D = 128
G = 64

NPAD = 10240
NC = 2
NS = 16
NW = NC * NS
CH = 125
NCHUNK = E // CH
CPT = NCHUNK // NW
ROWS_PER_TILE = NPAD // NS
EPT = E // NW

RB = 1024
GRID = NPAD // RB

_sc_mesh = plsc.VectorSubcoreMesh(core_axis_name="c", subcore_axis_name="s")

_sc_params = pltpu.CompilerParams()
if "needs_layout_passes" in pltpu.CompilerParams.__dataclass_fields__:
    _sc_params = dataclasses.replace(_sc_params, needs_layout_passes=False)


def _zero_tile_slice(acc, zbuf, base, nrows, sem):
    zr = zbuf.shape[0]

    @pl.loop(0, zr)
    def _(i):
        for j in range(zbuf.shape[1] // 16):
            zbuf[i, pl.ds(j * 16, 16)] = jnp.zeros((16,), jnp.float32)

    pltpu.async_copy(zbuf, acc.at[pl.ds(base, zr)], sem)
    pltpu.async_copy(zbuf, acc.at[pl.ds(base + zr, zr)], sem)

    @pl.loop(2, nrows // zr)
    def _(k):
        pltpu.make_async_copy(zbuf, acc.at[pl.ds(base, zr)], sem).wait()
        pltpu.async_copy(zbuf, acc.at[pl.ds(base + k * zr, zr)], sem)

    pltpu.make_async_copy(zbuf, acc.at[pl.ds(base, zr)], sem).wait()
    pltpu.make_async_copy(zbuf, acc.at[pl.ds(base, zr)], sem).wait()


def _sc_agg_body(h_hbm, src_hbm, dst_hbm, out_hbm, acc, zbuf, sidx, dring, rows,
                 sg0, sg1, ss0, ss1, si0, si1, sz):
    c = lax.axis_index("c")
    s = lax.axis_index("s")
    wid = s * NC + c
    cbase = wid * CPT

    pltpu.sync_copy(src_hbm.at[pl.ds(cbase, CPT)], sidx)
    pltpu.async_copy(dst_hbm.at[cbase], dring.at[0], si0)
    pltpu.async_copy(dst_hbm.at[cbase + 1], dring.at[1], si1)
    pltpu.async_copy(h_hbm.at[sidx.at[0]], rows.at[0], sg0)

    _zero_tile_slice(acc, zbuf, s * ROWS_PER_TILE, ROWS_PER_TILE, sz)
    plsc.subcore_barrier()

    sg = (sg0, sg1)
    ss = (ss0, ss1)
    si = (si0, si1)

    @pl.loop(0, CPT // 2)
    def _(i):
        for b in range(2):
            k = i * 2 + b
            r = lax.rem(k, 4)
            o = 1 - b
            pltpu.make_async_copy(dst_hbm.at[cbase], dring.at[r], si[b]).wait()
            if b == 0:
                @pl.when(i > 0)
                def _():
                    pltpu.make_async_copy(rows.at[o], acc.at[dring.at[r]], ss[o]).wait()

                pltpu.async_copy(h_hbm.at[sidx.at[k + 1]], rows.at[o], sg[o])
            else:
                pltpu.make_async_copy(rows.at[o], acc.at[dring.at[r]], ss[o]).wait()

                @pl.when(i < CPT // 2 - 1)
                def _():
                    pltpu.async_copy(h_hbm.at[sidx.at[k + 1]], rows.at[o], sg[o])

            pltpu.make_async_copy(h_hbm.at[sidx.at[k]], rows.at[b], sg[b]).wait()
            pltpu.async_copy(rows.at[b], acc.at[dring.at[r]], ss[b], add=True)

            @pl.when(k < CPT - 2)
            def _():
                pltpu.async_copy(dst_hbm.at[cbase + k + 2],
                                 dring.at[lax.rem(k + 2, 4)], si[b])

    pltpu.make_async_copy(rows.at[1], acc.at[dring.at[0]], ss1).wait()

    plsc.subcore_barrier()
    base = s * ROWS_PER_TILE
    pltpu.sync_copy(acc.at[pl.ds(base, ROWS_PER_TILE)],
                    out_hbm.at[c].at[pl.ds(base, ROWS_PER_TILE)])


def _sc_agg(h, src_r, dst_r):
    return pl.kernel(
        _sc_agg_body,
        out_type=jax.ShapeDtypeStruct((NC, NPAD, D), jnp.float32),
        mesh=_sc_mesh,
        scratch_types=[
            pltpu.VMEM_SHARED((NPAD, D), jnp.float32),
            pltpu.VMEM((32, D), jnp.float32),
            pltpu.VMEM((CPT, CH), jnp.int32),
            pltpu.VMEM((4, CH), jnp.int32),
            pltpu.VMEM((2, CH, D), jnp.float32),
            pltpu.SemaphoreType.DMA,
            pltpu.SemaphoreType.DMA,
            pltpu.SemaphoreType.DMA,
            pltpu.SemaphoreType.DMA,
            pltpu.SemaphoreType.DMA,
            pltpu.SemaphoreType.DMA,
            pltpu.SemaphoreType.DMA,
        ],
    )(h, src_r, dst_r)


def _sc_deg_body(dst_hbm, out_hbm, degt, didx):
    c = lax.axis_index("c")
    s = lax.axis_index("s")
    wid = s * NC + c

    pltpu.sync_copy(dst_hbm.at[pl.ds(wid * EPT, EPT)], didx)

    @pl.loop(0, NPAD // 128)
    def _(k):
        for j in range(8):
            degt[pl.ds(k * 128 + j * 16, 16)] = jnp.zeros((16,), jnp.float32)

    ones16 = jnp.ones((16,), jnp.float32)

    @pl.loop(0, EPT // 80)
    def _(i):
        for j in range(5):
            idx = didx[pl.ds(i * 80 + j * 16, 16)]
            plsc.addupdate_scatter(degt, [idx], ones16)

    pltpu.sync_copy(degt, out_hbm.at[wid])


def _sc_deg(dst):
    return pl.kernel(
        _sc_deg_body,
        out_type=jax.ShapeDtypeStruct((NW, NPAD), jnp.float32),
        mesh=_sc_mesh,
        scratch_types=[
            pltpu.VMEM((NPAD,), jnp.float32),
            pltpu.VMEM((EPT,), jnp.int32),
        ],
        compiler_params=_sc_params,
    )(dst)


def _tc_r_body(h_ref, wr_ref, bl_ref, o_ref):
    o_ref[...] = lax.dot_general(h_ref[...], wr_ref[...], (((1,), (0,)), ((), ())),
                                 precision=lax.Precision.HIGHEST,
                                 preferred_element_type=jnp.float32) + bl_ref[...]


def _tc_r(h, Wr, bl):
    return pl.pallas_call(
        _tc_r_body,
        grid=(GRID,),
        in_specs=[
            pl.BlockSpec((RB, D), lambda i: (i, 0)),
            pl.BlockSpec((D, D), lambda i: (0, 0)),
            pl.BlockSpec((1, D), lambda i: (0, 0)),
        ],
        out_specs=pl.BlockSpec((RB, D), lambda i: (i, 0)),
        out_shape=jax.ShapeDtypeStruct((NPAD, D), jnp.float32),
    )(h, Wr, bl.reshape(1, D))


def _scaled_agg_matmul(ap_ref, dp_ref, wl_ref):
    a = ap_ref[0] + ap_ref[1]
    deg = lax.dot_general(dp_ref[...], jnp.ones((NW, 1), jnp.float32),
                          (((0,), (0,)), ((), ())),
                          precision=lax.Precision.HIGHEST,
                          preferred_element_type=jnp.float32)
    inv = 1.0 / jnp.maximum(deg, 1.0)
    return lax.dot_general(a * inv, wl_ref[...], (((1,), (0,)), ((), ())),
                           precision=lax.Precision.HIGHEST,
                           preferred_element_type=jnp.float32)


def _tc_combine_body(ap_ref, dp_ref, r_ref, wl_ref, o_ref):
    out = _scaled_agg_matmul(ap_ref, dp_ref, wl_ref) + r_ref[...]
    o_ref[...] = jnp.maximum(out, 0.0)


def _tc_combine(parts, degp, r, Wl):
    return pl.pallas_call(
        _tc_combine_body,
        grid=(GRID,),
        in_specs=[
            pl.BlockSpec((NC, RB, D), lambda i: (0, i, 0)),
            pl.BlockSpec((NW, RB), lambda i: (0, i)),
            pl.BlockSpec((RB, D), lambda i: (i, 0)),
            pl.BlockSpec((D, D), lambda i: (0, 0)),
        ],
        out_specs=pl.BlockSpec((RB, D), lambda i: (i, 0)),
        out_shape=jax.ShapeDtypeStruct((NPAD, D), jnp.float32),
    )(parts, degp, r, Wl)


def _tc_combine_pool_body(ap_ref, dp_ref, r_ref, wl_ref, b_ref, o_ref,
                          s_acc, c_acc):
    i = pl.program_id(0)

    @pl.when(i == 0)
    def _():
        s_acc[...] = jnp.zeros_like(s_acc)
        c_acc[...] = jnp.zeros_like(c_acc)

    h4 = _scaled_agg_matmul(ap_ref, dp_ref, wl_ref) + r_ref[...]
    b = b_ref[0, 0, :]
    onehot = (lax.broadcasted_iota(jnp.int32, (G, RB), 0) == b[None, :]).astype(jnp.float32)
    s_acc[...] += lax.dot_general(onehot, h4, (((1,), (0,)), ((), ())),
                                  precision=lax.Precision.HIGHEST,
                                  preferred_element_type=jnp.float32)
    cnt = jnp.sum(onehot, axis=1, keepdims=True)
    c_acc[...] += jnp.broadcast_to(cnt, (G, D))

    @pl.when(i == GRID - 1)
    def _():
        o_ref[...] = s_acc[...] / jnp.maximum(c_acc[...], 1.0)


def _tc_combine_pool(parts, degp, r, Wl, batch_r):
    return pl.pallas_call(
        _tc_combine_pool_body,
        grid=(GRID,),
        in_specs=[
            pl.BlockSpec((NC, RB, D), lambda i: (0, i, 0)),
            pl.BlockSpec((NW, RB), lambda i: (0, i)),
            pl.BlockSpec((RB, D), lambda i: (i, 0)),
            pl.BlockSpec((D, D), lambda i: (0, 0)),
            pl.BlockSpec((1, 1, RB), lambda i: (i, 0, 0)),
        ],
        out_specs=pl.BlockSpec((G, D), lambda i: (0, 0)),
        out_shape=jax.ShapeDtypeStruct((G, D), jnp.float32),
        scratch_shapes=[
            pltpu.VMEM((G, D), jnp.float32),
            pltpu.VMEM((G, D), jnp.float32),
        ],
    )(parts, degp, r, Wl, batch_r)


def kernel(x, edge_index, batch, Wl1, bl1, Wr1, Wl2, bl2, Wr2, Wl3, bl3, Wr3,
           Wl4, bl4, Wr4):
    src_r = edge_index[0].reshape(NCHUNK, CH)
    dst_r = edge_index[1].reshape(NCHUNK, CH)
    h = jnp.pad(x, ((0, NPAD - N), (0, 0)))
    batch_r = jnp.pad(batch, (0, NPAD - N), constant_values=G).reshape(GRID, 1, RB)

    degp = _sc_deg(edge_index[1])

    for Wl, bl, Wr in ((Wl1, bl1, Wr1), (Wl2, bl2, Wr2), (Wl3, bl3, Wr3)):
        parts = _sc_agg(h, src_r, dst_r)
        r = _tc_r(h, Wr, bl)
        h = _tc_combine(parts, degp, r, Wl)

    parts = _sc_agg(h, src_r, dst_r)
    r = _tc_r(h, Wr4, bl4)
    return _tc_combine_pool(parts, degp, r, Wl4, batch_r)
